# baseline (speedup 1.0000x reference)
import functools

import jax
import jax.numpy as jnp
from jax import lax
from jax.experimental import pallas as pl
from jax.experimental.pallas import tpu as pltpu
from jax.experimental.pallas import tpu_sc as plsc

B = 64
N = 883
T = 12
TN = 16

B_SC = 32
B_TC = B - B_SC

NC = 2
NS = 16
NW = NC * NS

LN = 16
NPAD = 912
NCHUNK = 56
BP = B_SC // 2
JOBS = NCHUNK * BP
JPW = JOBS // NW
WIN = 48
BP_SHIFT = BP.bit_length() - 1
assert BP == 1 << BP_SHIFT

CSELF = float(0.12 ** 0.5)


def _tree_sum(xs):
    xs = list(xs)
    while len(xs) > 1:
        xs = [a + b for a, b in zip(xs[0::2], xs[1::2])] + (
            [xs[-1]] if len(xs) % 2 else [])
    return xs[0]


def _rsqrt(x, iters):
    xh = x * jnp.float32(0.5)
    i = lax.bitcast_convert_type(x, jnp.int32)
    i = jnp.int32(0x5F3759DF) - lax.shift_right_arithmetic(i, 1)
    y = lax.bitcast_convert_type(i, jnp.float32)
    for _ in range(iters):
        y = y * (jnp.float32(1.5) - xh * y * y)
    return y


def _sqrt(x, iters=3):
    return x * _rsqrt(jnp.maximum(x, jnp.float32(1e-30)), iters)


def _sc_body(enc_hbm, dec_hbm, k_hbm, v_hbm, aw_hbm, ab_hbm, out_hbm,
             enc_v, dec_v, k_v, v_v, aw_v, ab_v, in_sem, out_sem):
    wid = lax.axis_index("s") * NC + lax.axis_index("c")
    job0 = wid * JPW
    c0 = lax.shift_right_logical(job0, BP_SHIFT)
    n_lo = c0 * LN

    cps = [
        pltpu.async_copy(
            enc_hbm.at[pl.ds(0, B_SC), :, pl.ds(n_lo, WIN)], enc_v, in_sem),
        pltpu.async_copy(
            dec_hbm.at[pl.ds(0, B_SC), :, pl.ds(n_lo, WIN)], dec_v, in_sem),
        pltpu.async_copy(k_hbm.at[:, :, pl.ds(n_lo, WIN)], k_v, in_sem),
        pltpu.async_copy(v_hbm.at[:, :, pl.ds(n_lo, WIN)], v_v, in_sem),
        pltpu.async_copy(aw_hbm.at[:, pl.ds(n_lo, WIN)], aw_v, in_sem),
        pltpu.async_copy(ab_hbm.at[0, pl.ds(n_lo, WIN)], ab_v, in_sem),
    ]
    for cp in cps:
        cp.wait()

    cself = jnp.float32(CSELF)

    def body(i, carry):
        job = job0 + i
        chunk = lax.shift_right_logical(job, BP_SHIFT)
        b = (job - chunk * BP) * 2
        nloc = chunk * LN - n_lo
        ns = pl.ds(nloc, LN)

        q = [[enc_v[b + u, j, ns] for j in range(T)] for u in range(2)]

        d = [[], []]
        for t in range(TN):
            kt = [k_v[t, j, ns] for j in range(T)]
            for u in range(2):
                df0 = q[u][0] - kt[0]
                acc = df0 * df0
                for j in range(1, T):
                    df = q[u][j] - kt[j]
                    acc = acc + df * df
                d[u].append(_sqrt(acc, 2))

        e1 = [None, None]
        inv_z1 = [None, None]
        sum_d = [None, None]
        for u in range(2):
            sum_d[u] = _tree_sum(d[u])
            m1 = sum_d[u] * jnp.float32(1.0 / TN)
            dev = [m1 - dt for dt in d[u]]
            var1 = _tree_sum([x * x for x in dev]) * jnp.float32(
                1.0 / (TN - 1))
            std1 = _sqrt(var1) + jnp.float32(1e-6)
            coef1 = jnp.float32(10.0) / std1
            e1[u] = [jnp.exp(x * coef1) for x in dev]
            inv_z1[u] = jnp.float32(1.0) / _tree_sum(e1[u])

        att = [[], []]
        for j in range(T):
            vj = [v_v[t, j, ns] for t in range(TN)]
            for u in range(2):
                a = _tree_sum([e1[u][t] * vj[t] for t in range(TN)])
                att[u].append(a * inv_z1[u])

        awv = [aw_v[t, ns] for t in range(TN + 1)]
        abv = ab_v[ns]
        wgt = [None, None]
        for u in range(2):
            m2 = (sum_d[u] + cself) * jnp.float32(1.0 / (TN + 1))
            dev2 = [m2 - dt for dt in d[u]]
            dev2c = m2 - cself
            var2 = (_tree_sum([x * x for x in dev2]) + dev2c * dev2c) * (
                jnp.float32(1.0 / TN))
            std2 = _sqrt(var2) + jnp.float32(1e-6)
            coef2 = jnp.float32(10.0) / std2
            e2 = [jnp.exp(x * coef2) for x in dev2]
            e2c = jnp.exp(dev2c * coef2)
            z2 = _tree_sum(e2) + e2c
            num = _tree_sum([e2[t] * awv[t] for t in range(TN)])
            num = num + e2c * awv[TN]
            wgt[u] = num / z2 + abv

        for u in range(2):
            for j in range(T):
                dj = dec_v[b + u, j, ns]
                dec_v[b + u, j, ns] = att[u][j] + wgt[u] * (dj - att[u][j])

        pltpu.async_copy(dec_v.at[pl.ds(b, 2), :, pl.ds(nloc, LN)],
                         out_hbm.at[pl.ds(b, 2), :, pl.ds(chunk * LN, LN)],
                         out_sem)
        return carry

    lax.fori_loop(0, JPW, body, 0)

    def drain(i, carry):
        pltpu.make_async_copy(dec_v.at[pl.ds(0, 2), :, pl.ds(0, LN)],
                              out_hbm.at[pl.ds(0, 2), :, pl.ds(0, LN)],
                              out_sem).wait()
        return carry

    lax.fori_loop(0, JPW, drain, 0)


def _run_sc(enc_t, dec_t, k_t, v_t, aw_t, ab_t):
    mesh = plsc.VectorSubcoreMesh(
        core_axis_name="c", subcore_axis_name="s",
        num_cores=NC, num_subcores=NS)
    f = pl.kernel(
        _sc_body,
        out_type=jax.ShapeDtypeStruct((B_SC, T, NPAD), jnp.float32),
        mesh=mesh,
        compiler_params=pltpu.CompilerParams(use_tc_tiling_on_sc=False),
        scratch_types=[
            pltpu.VMEM((B_SC, T, WIN), jnp.float32),
            pltpu.VMEM((B_SC, T, WIN), jnp.float32),
            pltpu.VMEM((TN, T, WIN), jnp.float32),
            pltpu.VMEM((TN, T, WIN), jnp.float32),
            pltpu.VMEM((TN + 1, WIN), jnp.float32),
            pltpu.VMEM((WIN,), jnp.float32),
            pltpu.SemaphoreType.DMA,
            pltpu.SemaphoreType.DMA,
        ],
    )
    return f(enc_t, dec_t, k_t, v_t, aw_t, ab_t)


def _tc_body(enc_ref, dec_ref, k_ref, v_ref, aw_ref, ab_ref, out_ref):
    q = enc_ref[0]
    d = []
    for t in range(TN):
        diff = q - k_ref[t]
        d.append(jnp.sqrt(jnp.sum(diff * diff, axis=0)))
    dm = jnp.stack(d)
    sum_d = jnp.sum(dm, axis=0)
    m1 = sum_d * jnp.float32(1.0 / TN)
    dev = m1[None, :] - dm
    var1 = jnp.sum(dev * dev, axis=0) * jnp.float32(1.0 / (TN - 1))
    std1 = jnp.sqrt(var1) + jnp.float32(1e-6)
    coef1 = jnp.float32(10.0) / std1
    e1 = jnp.exp(dev * coef1[None, :])
    inv_z1 = jnp.float32(1.0) / jnp.sum(e1, axis=0)

    att = _tree_sum([e1[t][None, :] * v_ref[t] for t in range(TN)])
    att = att * inv_z1[None, :]

    cself = jnp.float32(CSELF)
    m2 = (sum_d + cself) * jnp.float32(1.0 / (TN + 1))
    dev2 = m2[None, :] - dm
    dev2c = m2 - cself
    var2 = (jnp.sum(dev2 * dev2, axis=0) + dev2c * dev2c) * jnp.float32(
        1.0 / TN)
    std2 = jnp.sqrt(var2) + jnp.float32(1e-6)
    coef2 = jnp.float32(10.0) / std2
    e2 = jnp.exp(dev2 * coef2[None, :])
    e2c = jnp.exp(dev2c * coef2)
    z2 = jnp.sum(e2, axis=0) + e2c
    num = jnp.sum(e2 * aw_ref[:TN], axis=0) + e2c * aw_ref[TN]
    w = num / z2 + ab_ref[0]

    dec = dec_ref[0]
    out_ref[0] = att + w[None, :] * (dec - att)


def _run_tc(enc_t, dec_t, k_t, v_t, aw_t, ab_t):
    return pl.pallas_call(
        _tc_body,
        grid=(B_TC,),
        in_specs=[
            pl.BlockSpec((1, T, NPAD), lambda b: (b + B_SC, 0, 0)),
            pl.BlockSpec((1, T, NPAD), lambda b: (b + B_SC, 0, 0)),
            pl.BlockSpec((TN, T, NPAD), lambda b: (0, 0, 0)),
            pl.BlockSpec((TN, T, NPAD), lambda b: (0, 0, 0)),
            pl.BlockSpec((TN + 1, NPAD), lambda b: (0, 0)),
            pl.BlockSpec((1, NPAD), lambda b: (0, 0)),
        ],
        out_specs=pl.BlockSpec((1, T, NPAD), lambda b: (b, 0, 0)),
        out_shape=jax.ShapeDtypeStruct((B_TC, T, NPAD), jnp.float32),
    )(enc_t, dec_t, k_t, v_t, aw_t, ab_t)


@jax.jit
def _run(enc_t, dec_t, k_t, v_t, aw_t, ab_t):
    sc_out = _run_sc(enc_t, dec_t, k_t, v_t, aw_t, ab_t)
    tc_out = _run_tc(enc_t, dec_t, k_t, v_t, aw_t, ab_t)
    return jnp.concatenate([sc_out, tc_out], axis=0)


def kernel(enc, x_mark_enc, dec, k, v, att_weight, att_bias):
    del x_mark_enc
    pad = NPAD - N
    enc_t = jnp.pad(jnp.transpose(enc, (0, 2, 1)), ((0, 0), (0, 0), (0, pad)))
    dec_t = jnp.pad(jnp.transpose(dec, (0, 2, 1)), ((0, 0), (0, 0), (0, pad)))
    k_t = jnp.pad(jnp.transpose(jnp.squeeze(k, 1), (1, 2, 0)),
                  ((0, 0), (0, 0), (0, pad)))
    v_t = jnp.pad(jnp.transpose(jnp.squeeze(v, 1), (1, 2, 0)),
                  ((0, 0), (0, 0), (0, pad)))
    aw_t = jnp.pad(att_weight.T, ((0, 0), (0, pad)))
    ab_t = jnp.pad(att_bias, ((0, pad)))[None, :]
    out_t = _run(enc_t, dec_t, k_t, v_t, aw_t, ab_t)
    return jnp.transpose(out_t[:, :, :N], (0, 2, 1))


# speedup vs baseline: 1.1024x; 1.1024x over previous
import functools

import jax
import jax.numpy as jnp
from jax import lax
from jax.experimental import pallas as pl
from jax.experimental.pallas import tpu as pltpu
from jax.experimental.pallas import tpu_sc as plsc

B = 64
N = 883
T = 12
TN = 16

B_SC = 32
B_TC = B - B_SC
TCB = 8

NC = 2
NS = 16
NW = NC * NS

LN = 16
NPAD = 912
NCHUNK = 56
BP = B_SC // 2
JOBS = NCHUNK * BP
JPW = JOBS // NW
WIN = 48
BP_SHIFT = BP.bit_length() - 1
assert BP == 1 << BP_SHIFT
assert B_TC % TCB == 0 and B_SC % TCB == 0

CSELF = float(0.12 ** 0.5)


def _tree_sum(xs):
    xs = list(xs)
    while len(xs) > 1:
        xs = [a + b for a, b in zip(xs[0::2], xs[1::2])] + (
            [xs[-1]] if len(xs) % 2 else [])
    return xs[0]


def _rsqrt(x, iters):
    xh = x * jnp.float32(0.5)
    i = lax.bitcast_convert_type(x, jnp.int32)
    i = jnp.int32(0x5F3759DF) - lax.shift_right_arithmetic(i, 1)
    y = lax.bitcast_convert_type(i, jnp.float32)
    for _ in range(iters):
        y = y * (jnp.float32(1.5) - xh * y * y)
    return y


def _sqrt(x, iters=3):
    return x * _rsqrt(jnp.maximum(x, jnp.float32(1e-30)), iters)


def _sc_body(enc_hbm, dec_hbm, k_hbm, v_hbm, aw_hbm, ab_hbm, out_hbm,
             enc_v, dec_v, k_v, v_v, aw_v, ab_v, in_sem, out_sem):
    wid = lax.axis_index("s") * NC + lax.axis_index("c")
    job0 = wid * JPW
    c0 = lax.shift_right_logical(job0, BP_SHIFT)
    n_lo = c0 * LN

    cps = [
        pltpu.async_copy(
            enc_hbm.at[:, pl.ds(0, B_SC), pl.ds(n_lo, WIN)], enc_v, in_sem),
        pltpu.async_copy(
            dec_hbm.at[:, pl.ds(0, B_SC), pl.ds(n_lo, WIN)], dec_v, in_sem),
        pltpu.async_copy(k_hbm.at[:, :, pl.ds(n_lo, WIN)], k_v, in_sem),
        pltpu.async_copy(v_hbm.at[:, :, pl.ds(n_lo, WIN)], v_v, in_sem),
        pltpu.async_copy(aw_hbm.at[:, pl.ds(n_lo, WIN)], aw_v, in_sem),
        pltpu.async_copy(ab_hbm.at[0, pl.ds(n_lo, WIN)], ab_v, in_sem),
    ]
    for cp in cps:
        cp.wait()

    cself = jnp.float32(CSELF)

    def body(i, carry):
        job = job0 + i
        chunk = lax.shift_right_logical(job, BP_SHIFT)
        b = (job - chunk * BP) * 2
        nloc = chunk * LN - n_lo
        ns = pl.ds(nloc, LN)

        q = [[enc_v[j, b + u, ns] for j in range(T)] for u in range(2)]

        d = [[], []]
        for t in range(TN):
            kt = [k_v[t, j, ns] for j in range(T)]
            for u in range(2):
                df0 = q[u][0] - kt[0]
                acc = df0 * df0
                for j in range(1, T):
                    df = q[u][j] - kt[j]
                    acc = acc + df * df
                d[u].append(_sqrt(acc, 2))

        e1 = [None, None]
        inv_z1 = [None, None]
        sum_d = [None, None]
        for u in range(2):
            sum_d[u] = _tree_sum(d[u])
            m1 = sum_d[u] * jnp.float32(1.0 / TN)
            dev = [m1 - dt for dt in d[u]]
            var1 = _tree_sum([x * x for x in dev]) * jnp.float32(
                1.0 / (TN - 1))
            std1 = _sqrt(var1) + jnp.float32(1e-6)
            coef1 = jnp.float32(10.0) / std1
            e1[u] = [jnp.exp(x * coef1) for x in dev]
            inv_z1[u] = jnp.float32(1.0) / _tree_sum(e1[u])

        att = [[], []]
        for j in range(T):
            vj = [v_v[t, j, ns] for t in range(TN)]
            for u in range(2):
                a = _tree_sum([e1[u][t] * vj[t] for t in range(TN)])
                att[u].append(a * inv_z1[u])

        awv = [aw_v[t, ns] for t in range(TN + 1)]
        abv = ab_v[ns]
        wgt = [None, None]
        for u in range(2):
            m2 = (sum_d[u] + cself) * jnp.float32(1.0 / (TN + 1))
            dev2 = [m2 - dt for dt in d[u]]
            dev2c = m2 - cself
            var2 = (_tree_sum([x * x for x in dev2]) + dev2c * dev2c) * (
                jnp.float32(1.0 / TN))
            std2 = _sqrt(var2) + jnp.float32(1e-6)
            coef2 = jnp.float32(10.0) / std2
            e2 = [jnp.exp(x * coef2) for x in dev2]
            e2c = jnp.exp(dev2c * coef2)
            z2 = _tree_sum(e2) + e2c
            num = _tree_sum([e2[t] * awv[t] for t in range(TN)])
            num = num + e2c * awv[TN]
            wgt[u] = num / z2 + abv

        for u in range(2):
            for j in range(T):
                dj = dec_v[j, b + u, ns]
                dec_v[j, b + u, ns] = att[u][j] + wgt[u] * (dj - att[u][j])

        pltpu.async_copy(dec_v.at[:, pl.ds(b, 2), pl.ds(nloc, LN)],
                         out_hbm.at[:, pl.ds(b, 2), pl.ds(chunk * LN, LN)],
                         out_sem)
        return carry

    lax.fori_loop(0, JPW, body, 0)

    def drain(i, carry):
        pltpu.make_async_copy(dec_v.at[:, pl.ds(0, 2), pl.ds(0, LN)],
                              out_hbm.at[:, pl.ds(0, 2), pl.ds(0, LN)],
                              out_sem).wait()
        return carry

    lax.fori_loop(0, JPW, drain, 0)


def _run_sc(enc_t, dec_t, k_t, v_t, aw_t, ab_t):
    mesh = plsc.VectorSubcoreMesh(
        core_axis_name="c", subcore_axis_name="s",
        num_cores=NC, num_subcores=NS)
    f = pl.kernel(
        _sc_body,
        out_type=jax.ShapeDtypeStruct((T, B_SC, NPAD), jnp.float32),
        mesh=mesh,
        compiler_params=pltpu.CompilerParams(use_tc_tiling_on_sc=False),
        scratch_types=[
            pltpu.VMEM((T, B_SC, WIN), jnp.float32),
            pltpu.VMEM((T, B_SC, WIN), jnp.float32),
            pltpu.VMEM((TN, T, WIN), jnp.float32),
            pltpu.VMEM((TN, T, WIN), jnp.float32),
            pltpu.VMEM((TN + 1, WIN), jnp.float32),
            pltpu.VMEM((WIN,), jnp.float32),
            pltpu.SemaphoreType.DMA,
            pltpu.SemaphoreType.DMA,
        ],
    )
    return f(enc_t, dec_t, k_t, v_t, aw_t, ab_t)


def _tc_body(enc_ref, dec_ref, k_ref, v_ref, aw_ref, ab_ref, out_ref):
    cself = jnp.float32(CSELF)
    q = [enc_ref[j] for j in range(T)]

    d = []
    for t in range(TN):
        kt = [k_ref[t, j][None, :] for j in range(T)]
        df0 = q[0] - kt[0]
        acc = df0 * df0
        for j in range(1, T):
            df = q[j] - kt[j]
            acc = acc + df * df
        d.append(jnp.sqrt(acc))

    sum_d = _tree_sum(d)
    m1 = sum_d * jnp.float32(1.0 / TN)
    dev = [m1 - dt for dt in d]
    var1 = _tree_sum([x * x for x in dev]) * jnp.float32(1.0 / (TN - 1))
    std1 = jnp.sqrt(var1) + jnp.float32(1e-6)
    coef1 = jnp.float32(10.0) / std1
    e1 = [jnp.exp(x * coef1) for x in dev]
    inv_z1 = jnp.float32(1.0) / _tree_sum(e1)

    att = []
    for j in range(T):
        a = _tree_sum([e1[t] * v_ref[t, j][None, :] for t in range(TN)])
        att.append(a * inv_z1)

    m2 = (sum_d + cself) * jnp.float32(1.0 / (TN + 1))
    dev2 = [m2 - dt for dt in d]
    dev2c = m2 - cself
    var2 = (_tree_sum([x * x for x in dev2]) + dev2c * dev2c) * jnp.float32(
        1.0 / TN)
    std2 = jnp.sqrt(var2) + jnp.float32(1e-6)
    coef2 = jnp.float32(10.0) / std2
    e2 = [jnp.exp(x * coef2) for x in dev2]
    e2c = jnp.exp(dev2c * coef2)
    z2 = _tree_sum(e2) + e2c
    num = _tree_sum([e2[t] * aw_ref[t][None, :] for t in range(TN)])
    num = num + e2c * aw_ref[TN][None, :]
    w = num / z2 + ab_ref[0][None, :]

    for j in range(T):
        dj = dec_ref[j]
        out_ref[j] = att[j] + w * (dj - att[j])


def _run_tc(enc_t, dec_t, k_t, v_t, aw_t, ab_t):
    nb = B_SC // TCB
    return pl.pallas_call(
        _tc_body,
        grid=(B_TC // TCB,),
        in_specs=[
            pl.BlockSpec((T, TCB, NPAD), lambda g: (0, g + nb, 0)),
            pl.BlockSpec((T, TCB, NPAD), lambda g: (0, g + nb, 0)),
            pl.BlockSpec((TN, T, NPAD), lambda g: (0, 0, 0)),
            pl.BlockSpec((TN, T, NPAD), lambda g: (0, 0, 0)),
            pl.BlockSpec((TN + 1, NPAD), lambda g: (0, 0)),
            pl.BlockSpec((1, NPAD), lambda g: (0, 0)),
        ],
        out_specs=pl.BlockSpec((T, TCB, NPAD), lambda g: (0, g, 0)),
        out_shape=jax.ShapeDtypeStruct((T, B_TC, NPAD), jnp.float32),
    )(enc_t, dec_t, k_t, v_t, aw_t, ab_t)


@jax.jit
def _run(enc_t, dec_t, k_t, v_t, aw_t, ab_t):
    sc_out = _run_sc(enc_t, dec_t, k_t, v_t, aw_t, ab_t)
    tc_out = _run_tc(enc_t, dec_t, k_t, v_t, aw_t, ab_t)
    return jnp.concatenate([sc_out, tc_out], axis=1)


def kernel(enc, x_mark_enc, dec, k, v, att_weight, att_bias):
    del x_mark_enc
    pad = NPAD - N
    enc_t = jnp.pad(jnp.transpose(enc, (2, 0, 1)), ((0, 0), (0, 0), (0, pad)))
    dec_t = jnp.pad(jnp.transpose(dec, (2, 0, 1)), ((0, 0), (0, 0), (0, pad)))
    k_t = jnp.pad(jnp.transpose(jnp.squeeze(k, 1), (1, 2, 0)),
                  ((0, 0), (0, 0), (0, pad)))
    v_t = jnp.pad(jnp.transpose(jnp.squeeze(v, 1), (1, 2, 0)),
                  ((0, 0), (0, 0), (0, pad)))
    aw_t = jnp.pad(att_weight.T, ((0, 0), (0, pad)))
    ab_t = jnp.pad(att_bias, ((0, pad)))[None, :]
    out_t = _run(enc_t, dec_t, k_t, v_t, aw_t, ab_t)
    return jnp.transpose(out_t[:, :, :N], (1, 2, 0))


# speedup vs baseline: 1.2387x; 1.1237x over previous
import functools

import jax
import jax.numpy as jnp
from jax import lax
from jax.experimental import pallas as pl
from jax.experimental.pallas import tpu as pltpu
from jax.experimental.pallas import tpu_sc as plsc

B = 64
N = 883
T = 12
TN = 16

B_SC = 16
B_TC = B - B_SC
TCB = 8

NC = 2
NS = 16
NW = NC * NS

LN = 16
NPAD = 912
NCHUNK = 56
BP = B_SC // 2
JOBS = NCHUNK * BP
JPW = JOBS // NW
WIN = 48
BP_SHIFT = BP.bit_length() - 1
assert BP == 1 << BP_SHIFT
assert B_TC % TCB == 0 and B_SC % TCB == 0

CSELF = float(0.12 ** 0.5)


def _tree_sum(xs):
    xs = list(xs)
    while len(xs) > 1:
        xs = [a + b for a, b in zip(xs[0::2], xs[1::2])] + (
            [xs[-1]] if len(xs) % 2 else [])
    return xs[0]


def _rsqrt(x, iters):
    xh = x * jnp.float32(0.5)
    i = lax.bitcast_convert_type(x, jnp.int32)
    i = jnp.int32(0x5F3759DF) - lax.shift_right_arithmetic(i, 1)
    y = lax.bitcast_convert_type(i, jnp.float32)
    for _ in range(iters):
        y = y * (jnp.float32(1.5) - xh * y * y)
    return y


def _sqrt(x, iters=3):
    return x * _rsqrt(jnp.maximum(x, jnp.float32(1e-30)), iters)


def _sc_body(enc_hbm, dec_hbm, k_hbm, v_hbm, aw_hbm, ab_hbm, out_hbm,
             enc_v, dec_v, k_v, v_v, aw_v, ab_v, in_sem, out_sem):
    wid = lax.axis_index("s") * NC + lax.axis_index("c")
    job0 = wid * JPW
    c0 = lax.shift_right_logical(job0, BP_SHIFT)
    n_lo = c0 * LN

    cps = [
        pltpu.async_copy(
            enc_hbm.at[:, pl.ds(0, B_SC), pl.ds(n_lo, WIN)], enc_v, in_sem),
        pltpu.async_copy(
            dec_hbm.at[:, pl.ds(0, B_SC), pl.ds(n_lo, WIN)], dec_v, in_sem),
        pltpu.async_copy(k_hbm.at[:, :, pl.ds(n_lo, WIN)], k_v, in_sem),
        pltpu.async_copy(v_hbm.at[:, :, pl.ds(n_lo, WIN)], v_v, in_sem),
        pltpu.async_copy(aw_hbm.at[:, pl.ds(n_lo, WIN)], aw_v, in_sem),
        pltpu.async_copy(ab_hbm.at[0, pl.ds(n_lo, WIN)], ab_v, in_sem),
    ]
    for cp in cps:
        cp.wait()

    cself = jnp.float32(CSELF)

    def body(i, carry):
        job = job0 + i
        chunk = lax.shift_right_logical(job, BP_SHIFT)
        b = (job - chunk * BP) * 2
        nloc = chunk * LN - n_lo
        ns = pl.ds(nloc, LN)

        q = [[enc_v[j, b + u, ns] for j in range(T)] for u in range(2)]

        d = [[], []]
        for t in range(TN):
            kt = [k_v[t, j, ns] for j in range(T)]
            for u in range(2):
                df0 = q[u][0] - kt[0]
                acc = df0 * df0
                for j in range(1, T):
                    df = q[u][j] - kt[j]
                    acc = acc + df * df
                d[u].append(_sqrt(acc, 2))

        e1 = [None, None]
        inv_z1 = [None, None]
        sum_d = [None, None]
        for u in range(2):
            sum_d[u] = _tree_sum(d[u])
            m1 = sum_d[u] * jnp.float32(1.0 / TN)
            dev = [m1 - dt for dt in d[u]]
            var1 = _tree_sum([x * x for x in dev]) * jnp.float32(
                1.0 / (TN - 1))
            std1 = _sqrt(var1) + jnp.float32(1e-6)
            coef1 = jnp.float32(10.0) / std1
            e1[u] = [jnp.exp(x * coef1) for x in dev]
            inv_z1[u] = jnp.float32(1.0) / _tree_sum(e1[u])

        att = [[], []]
        for j in range(T):
            vj = [v_v[t, j, ns] for t in range(TN)]
            for u in range(2):
                a = _tree_sum([e1[u][t] * vj[t] for t in range(TN)])
                att[u].append(a * inv_z1[u])

        awv = [aw_v[t, ns] for t in range(TN + 1)]
        abv = ab_v[ns]
        wgt = [None, None]
        for u in range(2):
            m2 = (sum_d[u] + cself) * jnp.float32(1.0 / (TN + 1))
            dev2 = [m2 - dt for dt in d[u]]
            dev2c = m2 - cself
            var2 = (_tree_sum([x * x for x in dev2]) + dev2c * dev2c) * (
                jnp.float32(1.0 / TN))
            std2 = _sqrt(var2) + jnp.float32(1e-6)
            coef2 = jnp.float32(10.0) / std2
            e2 = [jnp.exp(x * coef2) for x in dev2]
            e2c = jnp.exp(dev2c * coef2)
            z2 = _tree_sum(e2) + e2c
            num = _tree_sum([e2[t] * awv[t] for t in range(TN)])
            num = num + e2c * awv[TN]
            wgt[u] = num / z2 + abv

        for u in range(2):
            for j in range(T):
                dj = dec_v[j, b + u, ns]
                dec_v[j, b + u, ns] = att[u][j] + wgt[u] * (dj - att[u][j])

        pltpu.async_copy(dec_v.at[:, pl.ds(b, 2), pl.ds(nloc, LN)],
                         out_hbm.at[:, pl.ds(b, 2), pl.ds(chunk * LN, LN)],
                         out_sem)
        return carry

    lax.fori_loop(0, JPW, body, 0)

    def drain(i, carry):
        pltpu.make_async_copy(dec_v.at[:, pl.ds(0, 2), pl.ds(0, LN)],
                              out_hbm.at[:, pl.ds(0, 2), pl.ds(0, LN)],
                              out_sem).wait()
        return carry

    lax.fori_loop(0, JPW, drain, 0)


def _run_sc(enc_t, dec_t, k_t, v_t, aw_t, ab_t):
    mesh = plsc.VectorSubcoreMesh(
        core_axis_name="c", subcore_axis_name="s",
        num_cores=NC, num_subcores=NS)
    f = pl.kernel(
        _sc_body,
        out_type=jax.ShapeDtypeStruct((T, B_SC, NPAD), jnp.float32),
        mesh=mesh,
        compiler_params=pltpu.CompilerParams(use_tc_tiling_on_sc=False),
        scratch_types=[
            pltpu.VMEM((T, B_SC, WIN), jnp.float32),
            pltpu.VMEM((T, B_SC, WIN), jnp.float32),
            pltpu.VMEM((TN, T, WIN), jnp.float32),
            pltpu.VMEM((TN, T, WIN), jnp.float32),
            pltpu.VMEM((TN + 1, WIN), jnp.float32),
            pltpu.VMEM((WIN,), jnp.float32),
            pltpu.SemaphoreType.DMA,
            pltpu.SemaphoreType.DMA,
        ],
    )
    return f(enc_t, dec_t, k_t, v_t, aw_t, ab_t)


def _tc_body(enc_ref, dec_ref, k_ref, v_ref, aw_ref, ab_ref, out_ref):
    cself = jnp.float32(CSELF)
    q = [enc_ref[j] for j in range(T)]

    d = []
    for t in range(TN):
        kt = [k_ref[t, j][None, :] for j in range(T)]
        df0 = q[0] - kt[0]
        acc = df0 * df0
        for j in range(1, T):
            df = q[j] - kt[j]
            acc = acc + df * df
        d.append(jnp.sqrt(acc))

    sum_d = _tree_sum(d)
    m1 = sum_d * jnp.float32(1.0 / TN)
    dev = [m1 - dt for dt in d]
    var1 = _tree_sum([x * x for x in dev]) * jnp.float32(1.0 / (TN - 1))
    std1 = jnp.sqrt(var1) + jnp.float32(1e-6)
    coef1 = jnp.float32(10.0) / std1
    e1 = [jnp.exp(x * coef1) for x in dev]
    inv_z1 = jnp.float32(1.0) / _tree_sum(e1)

    att = []
    for j in range(T):
        a = _tree_sum([e1[t] * v_ref[t, j][None, :] for t in range(TN)])
        att.append(a * inv_z1)

    m2 = (sum_d + cself) * jnp.float32(1.0 / (TN + 1))
    dev2 = [m2 - dt for dt in d]
    dev2c = m2 - cself
    var2 = (_tree_sum([x * x for x in dev2]) + dev2c * dev2c) * jnp.float32(
        1.0 / TN)
    std2 = jnp.sqrt(var2) + jnp.float32(1e-6)
    coef2 = jnp.float32(10.0) / std2
    e2 = [jnp.exp(x * coef2) for x in dev2]
    e2c = jnp.exp(dev2c * coef2)
    z2 = _tree_sum(e2) + e2c
    num = _tree_sum([e2[t] * aw_ref[t][None, :] for t in range(TN)])
    num = num + e2c * aw_ref[TN][None, :]
    w = num / z2 + ab_ref[0][None, :]

    for j in range(T):
        dj = dec_ref[j]
        out_ref[j] = att[j] + w * (dj - att[j])


def _run_tc(enc_t, dec_t, k_t, v_t, aw_t, ab_t):
    nb = B_SC // TCB
    return pl.pallas_call(
        _tc_body,
        grid=(B_TC // TCB,),
        in_specs=[
            pl.BlockSpec((T, TCB, NPAD), lambda g: (0, g + nb, 0)),
            pl.BlockSpec((T, TCB, NPAD), lambda g: (0, g + nb, 0)),
            pl.BlockSpec((TN, T, NPAD), lambda g: (0, 0, 0)),
            pl.BlockSpec((TN, T, NPAD), lambda g: (0, 0, 0)),
            pl.BlockSpec((TN + 1, NPAD), lambda g: (0, 0)),
            pl.BlockSpec((1, NPAD), lambda g: (0, 0)),
        ],
        out_specs=pl.BlockSpec((T, TCB, NPAD), lambda g: (0, g, 0)),
        out_shape=jax.ShapeDtypeStruct((T, B_TC, NPAD), jnp.float32),
    )(enc_t, dec_t, k_t, v_t, aw_t, ab_t)


@jax.jit
def _run(enc_t, dec_t, k_t, v_t, aw_t, ab_t):
    sc_out = _run_sc(enc_t, dec_t, k_t, v_t, aw_t, ab_t)
    tc_out = _run_tc(enc_t, dec_t, k_t, v_t, aw_t, ab_t)
    return jnp.concatenate([sc_out, tc_out], axis=1)


def kernel(enc, x_mark_enc, dec, k, v, att_weight, att_bias):
    del x_mark_enc
    pad = NPAD - N
    enc_t = jnp.pad(jnp.transpose(enc, (2, 0, 1)), ((0, 0), (0, 0), (0, pad)))
    dec_t = jnp.pad(jnp.transpose(dec, (2, 0, 1)), ((0, 0), (0, 0), (0, pad)))
    k_t = jnp.pad(jnp.transpose(jnp.squeeze(k, 1), (1, 2, 0)),
                  ((0, 0), (0, 0), (0, pad)))
    v_t = jnp.pad(jnp.transpose(jnp.squeeze(v, 1), (1, 2, 0)),
                  ((0, 0), (0, 0), (0, pad)))
    aw_t = jnp.pad(att_weight.T, ((0, 0), (0, pad)))
    ab_t = jnp.pad(att_bias, ((0, pad)))[None, :]
    out_t = _run(enc_t, dec_t, k_t, v_t, aw_t, ab_t)
    return jnp.transpose(out_t[:, :, :N], (1, 2, 0))
